# baseline (speedup 1.0000x reference)
import functools

import jax
import jax.numpy as jnp
from jax import lax
from jax.experimental import pallas as pl
from jax.experimental.pallas import tpu as pltpu
from jax.experimental.pallas import tpu_sc as plsc

E = 16
TOPK = 2
T = 2048
D = 768
F = 1536
P = T * TOPK
TILE = 384
NT = P // TILE + (E - 1)
NTPAD = 32
CF = 256
C = F // CF
NP = NT * TILE
NW = 32
TPW = T // NW
LANES = 16


def _routing_body(logits_ref, slot0_ref, slot1_ref, w0_ref, w1_ref,
                  eot_ref, ntot_ref):
    logits = logits_ref[...]
    m = jnp.max(logits, axis=1, keepdims=True)
    ex = jnp.exp(logits - m)
    probs = ex / jnp.sum(ex, axis=1, keepdims=True)

    ie = lax.broadcasted_iota(jnp.int32, (T, E), 1)
    m1 = jnp.max(probs, axis=1, keepdims=True)
    a1 = jnp.min(jnp.where(probs >= m1, ie, E), axis=1, keepdims=True)
    oh1 = (ie == a1)
    pmask = jnp.where(oh1, -1.0, probs)
    m2 = jnp.max(pmask, axis=1, keepdims=True)
    a2 = jnp.min(jnp.where(pmask >= m2, ie, E), axis=1, keepdims=True)
    oh2 = (ie == a2)
    rs = m1 + m2
    w0_ref[...] = jnp.broadcast_to(m1 / rs, (T, LANES))
    w1_ref[...] = jnp.broadcast_to(m2 / rs, (T, LANES))

    oh1f = oh1.astype(jnp.float32)
    oh2f = oh2.astype(jnp.float32)
    bb = oh1f + oh2f
    tri = (lax.broadcasted_iota(jnp.int32, (128, 128), 0)
           > lax.broadcasted_iota(jnp.int32, (128, 128), 1)).astype(jnp.float32)
    run = jnp.zeros((1, E), jnp.float32)
    blocks = []
    for b in range(T // 128):
        blk = bb[b * 128:(b + 1) * 128, :]
        pref = lax.dot_general(tri, blk, (((1,), (0,)), ((), ())),
                               preferred_element_type=jnp.float32)
        blocks.append(pref + run)
        run = run + jnp.sum(blk, axis=0, keepdims=True)
    cex = jnp.concatenate(blocks, axis=0)
    cnt = run.astype(jnp.int32)
    ntile = (cnt + (TILE - 1)) // TILE
    upper = (lax.broadcasted_iota(jnp.int32, (E, E), 0)
             < lax.broadcasted_iota(jnp.int32, (E, E), 1)).astype(jnp.float32)
    ts = lax.dot_general(ntile.astype(jnp.float32), upper,
                         (((1,), (0,)), ((), ())),
                         preferred_element_type=jnp.float32)
    seg = ts * TILE
    total = jnp.sum(ntile)
    slot0_ref[...] = jnp.sum(oh1f * (cex + seg), axis=1).astype(jnp.int32)
    slot1_ref[...] = jnp.sum(oh2f * (cex + seg), axis=1).astype(jnp.int32)
    jm = lax.broadcasted_iota(jnp.int32, (NTPAD, E), 0)
    jj = jnp.minimum(jm, total - 1)
    tsi = ts.astype(jnp.int32)
    eot_ref[...] = jnp.sum((jj >= tsi).astype(jnp.int32), axis=1) - 1
    ntot_ref[0] = total


def _routing(router_logits):
    return pl.pallas_call(
        _routing_body,
        out_shape=(
            jax.ShapeDtypeStruct((T,), jnp.int32),
            jax.ShapeDtypeStruct((T,), jnp.int32),
            jax.ShapeDtypeStruct((T, LANES), jnp.float32),
            jax.ShapeDtypeStruct((T, LANES), jnp.float32),
            jax.ShapeDtypeStruct((NTPAD,), jnp.int32),
            jax.ShapeDtypeStruct((1,), jnp.int32),
        ),
        out_specs=(
            pl.BlockSpec((T,), lambda: (0,)),
            pl.BlockSpec((T,), lambda: (0,)),
            pl.BlockSpec((T, LANES), lambda: (0, 0)),
            pl.BlockSpec((T, LANES), lambda: (0, 0)),
            pl.BlockSpec((NTPAD,), lambda: (0,)),
            pl.BlockSpec(memory_space=pltpu.SMEM),
        ),
    )(router_logits)


def _dispatch_body(h_hbm, s0_hbm, s1_hbm, xs_hbm, idx0, idx1, xbuf,
                   sem0, sem1):
    wid = lax.axis_index("c") * (NW // 2) + lax.axis_index("s")
    base = wid * TPW
    pltpu.sync_copy(s0_hbm.at[pl.ds(base, TPW)], idx0)
    pltpu.sync_copy(s1_hbm.at[pl.ds(base, TPW)], idx1)
    pltpu.sync_copy(h_hbm.at[pl.ds(base, TPW)], xbuf)
    c0 = pltpu.async_copy(xbuf, xs_hbm.at[idx0], sem0)
    c1 = pltpu.async_copy(xbuf, xs_hbm.at[idx1], sem1)
    c0.wait()
    c1.wait()


@functools.cache
def _make_dispatch():
    return pl.kernel(
        _dispatch_body,
        out_type=jax.ShapeDtypeStruct((NP, D), jnp.float32),
        mesh=plsc.VectorSubcoreMesh(core_axis_name="c", subcore_axis_name="s"),
        scratch_types=[
            pltpu.VMEM((TPW,), jnp.int32),
            pltpu.VMEM((TPW,), jnp.int32),
            pltpu.VMEM((TPW, D), jnp.float32),
            pltpu.SemaphoreType.DMA,
            pltpu.SemaphoreType.DMA,
        ],
    )


def _gemm_body(eot_ref, ntot_ref, x_ref, wg0_ref, wu0_ref, w20_ref,
               wg1_ref, wu1_ref, w21_ref, o_ref):
    i = pl.program_id(0)
    c = pl.program_id(1)

    @pl.when(i < ntot_ref[0])
    def _():
        x = x_ref[...]
        acc = None
        for wg_ref, wu_ref, w2_ref in ((wg0_ref, wu0_ref, w20_ref),
                                       (wg1_ref, wu1_ref, w21_ref)):
            gate = lax.dot_general(x, wg_ref[0], (((1,), (1,)), ((), ())),
                                   preferred_element_type=jnp.float32)
            up = lax.dot_general(x, wu_ref[0], (((1,), (1,)), ((), ())),
                                 preferred_element_type=jnp.float32)
            h = gate * jax.nn.sigmoid(gate) * up
            part = lax.dot_general(h, w2_ref[0], (((1,), (1,)), ((), ())),
                                   preferred_element_type=jnp.float32)
            acc = part if acc is None else acc + part

        @pl.when(c == 0)
        def _():
            o_ref[...] = acc

        @pl.when(c > 0)
        def _():
            o_ref[...] += acc


def _gemm(x_sorted, w13_weight, w2_weight, eot, ntot):
    h = CF // 2
    nh = F // h

    def _ce(i, c, ntot):
        return jnp.where(i < ntot[0], c, C - 1)

    def _wg(s):
        return pl.BlockSpec(
            (1, h, D), lambda i, c, eot, ntot: (eot[i], 2 * _ce(i, c, ntot) + s, 0))

    def _wu(s):
        return pl.BlockSpec(
            (1, h, D),
            lambda i, c, eot, ntot: (eot[i], nh + 2 * _ce(i, c, ntot) + s, 0))

    def _w2(s):
        return pl.BlockSpec(
            (1, D, h), lambda i, c, eot, ntot: (eot[i], 0, 2 * _ce(i, c, ntot) + s))

    grid_spec = pltpu.PrefetchScalarGridSpec(
        num_scalar_prefetch=2,
        grid=(NT, C),
        in_specs=[
            pl.BlockSpec((TILE, D),
                         lambda i, c, eot, ntot: (jnp.minimum(i, ntot[0] - 1), 0)),
            _wg(0), _wu(0), _w2(0),
            _wg(1), _wu(1), _w2(1),
        ],
        out_specs=pl.BlockSpec((TILE, D), lambda i, c, eot, ntot: (i, 0)),
    )
    return pl.pallas_call(
        _gemm_body,
        grid_spec=grid_spec,
        out_shape=jax.ShapeDtypeStruct((NP, D), jnp.float32),
        compiler_params=pltpu.CompilerParams(
            dimension_semantics=("arbitrary", "arbitrary"),
            vmem_limit_bytes=100 * 1024 * 1024,
        ),
    )(eot, ntot, x_sorted, w13_weight, w13_weight, w2_weight,
      w13_weight, w13_weight, w2_weight)


def _combine_body(os_hbm, s0_hbm, s1_hbm, w0_hbm, w1_hbm, out_hbm,
                  idx0, idx1, wv0, wv1, buf_a, buf_b, sem_a, sem_b):
    wid = lax.axis_index("c") * (NW // 2) + lax.axis_index("s")
    base = wid * TPW
    pltpu.sync_copy(s0_hbm.at[pl.ds(base, TPW)], idx0)
    pltpu.sync_copy(s1_hbm.at[pl.ds(base, TPW)], idx1)
    pltpu.sync_copy(w0_hbm.at[pl.ds(base, TPW)], wv0)
    pltpu.sync_copy(w1_hbm.at[pl.ds(base, TPW)], wv1)
    ca = pltpu.async_copy(os_hbm.at[idx0], buf_a, sem_a)
    cb = pltpu.async_copy(os_hbm.at[idx1], buf_b, sem_b)
    ca.wait()
    cb.wait()

    def row(r, carry):
        wa = wv0[r, :]
        wb = wv1[r, :]
        for c in range(D // LANES):
            sl = pl.ds(c * LANES, LANES)
            buf_a[r, sl] = wa * buf_a[r, sl] + wb * buf_b[r, sl]
        return carry

    lax.fori_loop(0, TPW, row, 0)
    pltpu.sync_copy(buf_a, out_hbm.at[pl.ds(base, TPW)])


@functools.cache
def _make_combine():
    return pl.kernel(
        _combine_body,
        out_type=jax.ShapeDtypeStruct((T, D), jnp.float32),
        mesh=plsc.VectorSubcoreMesh(core_axis_name="c", subcore_axis_name="s"),
        scratch_types=[
            pltpu.VMEM((TPW,), jnp.int32),
            pltpu.VMEM((TPW,), jnp.int32),
            pltpu.VMEM((TPW, LANES), jnp.float32),
            pltpu.VMEM((TPW, LANES), jnp.float32),
            pltpu.VMEM((TPW, D), jnp.float32),
            pltpu.VMEM((TPW, D), jnp.float32),
            pltpu.SemaphoreType.DMA,
            pltpu.SemaphoreType.DMA,
        ],
    )


def kernel(hidden_states, router_logits, w13_weight, w2_weight):
    slot0, slot1, w0, w1, eot, ntot = _routing(router_logits)
    x_sorted = _make_dispatch()(hidden_states, slot0, slot1)
    o_sorted = _gemm(x_sorted, w13_weight, w2_weight, eot, ntot)
    return _make_combine()(o_sorted, slot0, slot1, w0, w1)


# speedup vs baseline: 1.3984x; 1.3984x over previous
import functools

import jax
import jax.numpy as jnp
from jax import lax
from jax.experimental import pallas as pl
from jax.experimental.pallas import tpu as pltpu
from jax.experimental.pallas import tpu_sc as plsc

E = 16
TOPK = 2
T = 2048
D = 768
F = 1536
P = T * TOPK
TILE = 384
NT = P // TILE + (E - 1)
NTPAD = 32
CF = 512
C = F // CF
NP = NT * TILE
NW = 32
TPW = T // NW
LANES = 16


def _routing_body(logits_ref, slot0_ref, slot1_ref, w0_ref, w1_ref,
                  eot_ref, ntot_ref):
    logits = logits_ref[...]
    m = jnp.max(logits, axis=1, keepdims=True)
    ex = jnp.exp(logits - m)
    probs = ex / jnp.sum(ex, axis=1, keepdims=True)

    ie = lax.broadcasted_iota(jnp.int32, (T, E), 1)
    m1 = jnp.max(probs, axis=1, keepdims=True)
    a1 = jnp.min(jnp.where(probs >= m1, ie, E), axis=1, keepdims=True)
    oh1 = (ie == a1)
    pmask = jnp.where(oh1, -1.0, probs)
    m2 = jnp.max(pmask, axis=1, keepdims=True)
    a2 = jnp.min(jnp.where(pmask >= m2, ie, E), axis=1, keepdims=True)
    oh2 = (ie == a2)
    rs = m1 + m2
    w0_ref[...] = jnp.broadcast_to(m1 / rs, (T, LANES))
    w1_ref[...] = jnp.broadcast_to(m2 / rs, (T, LANES))

    oh1f = oh1.astype(jnp.float32)
    oh2f = oh2.astype(jnp.float32)
    bb = oh1f + oh2f
    tri = (lax.broadcasted_iota(jnp.int32, (128, 128), 0)
           > lax.broadcasted_iota(jnp.int32, (128, 128), 1)).astype(jnp.float32)
    run = jnp.zeros((1, E), jnp.float32)
    blocks = []
    for b in range(T // 128):
        blk = bb[b * 128:(b + 1) * 128, :]
        pref = lax.dot_general(tri, blk, (((1,), (0,)), ((), ())),
                               preferred_element_type=jnp.float32)
        blocks.append(pref + run)
        run = run + jnp.sum(blk, axis=0, keepdims=True)
    cex = jnp.concatenate(blocks, axis=0)
    cnt = run.astype(jnp.int32)
    ntile = (cnt + (TILE - 1)) // TILE
    upper = (lax.broadcasted_iota(jnp.int32, (E, E), 0)
             < lax.broadcasted_iota(jnp.int32, (E, E), 1)).astype(jnp.float32)
    ts = lax.dot_general(ntile.astype(jnp.float32), upper,
                         (((1,), (0,)), ((), ())),
                         preferred_element_type=jnp.float32)
    seg = ts * TILE
    total = jnp.sum(ntile)
    slot0_ref[...] = jnp.sum(oh1f * (cex + seg), axis=1).astype(jnp.int32)
    slot1_ref[...] = jnp.sum(oh2f * (cex + seg), axis=1).astype(jnp.int32)
    jm = lax.broadcasted_iota(jnp.int32, (NTPAD, E), 0)
    jj = jnp.minimum(jm, total - 1)
    tsi = ts.astype(jnp.int32)
    eot_ref[...] = jnp.sum((jj >= tsi).astype(jnp.int32), axis=1) - 1
    ntot_ref[0] = total


def _routing(router_logits):
    return pl.pallas_call(
        _routing_body,
        out_shape=(
            jax.ShapeDtypeStruct((T,), jnp.int32),
            jax.ShapeDtypeStruct((T,), jnp.int32),
            jax.ShapeDtypeStruct((T, LANES), jnp.float32),
            jax.ShapeDtypeStruct((T, LANES), jnp.float32),
            jax.ShapeDtypeStruct((NTPAD,), jnp.int32),
            jax.ShapeDtypeStruct((1,), jnp.int32),
        ),
        out_specs=(
            pl.BlockSpec((T,), lambda: (0,)),
            pl.BlockSpec((T,), lambda: (0,)),
            pl.BlockSpec((T, LANES), lambda: (0, 0)),
            pl.BlockSpec((T, LANES), lambda: (0, 0)),
            pl.BlockSpec((NTPAD,), lambda: (0,)),
            pl.BlockSpec(memory_space=pltpu.SMEM),
        ),
    )(router_logits)


def _dispatch_body(h_hbm, s0_hbm, s1_hbm, xs_hbm, idx0, idx1, xbuf,
                   sem0, sem1):
    wid = lax.axis_index("c") * (NW // 2) + lax.axis_index("s")
    base = wid * TPW
    pltpu.sync_copy(s0_hbm.at[pl.ds(base, TPW)], idx0)
    pltpu.sync_copy(s1_hbm.at[pl.ds(base, TPW)], idx1)
    pltpu.sync_copy(h_hbm.at[pl.ds(base, TPW)], xbuf)
    c0 = pltpu.async_copy(xbuf, xs_hbm.at[idx0], sem0)
    c1 = pltpu.async_copy(xbuf, xs_hbm.at[idx1], sem1)
    c0.wait()
    c1.wait()


@functools.cache
def _make_dispatch():
    return pl.kernel(
        _dispatch_body,
        out_type=jax.ShapeDtypeStruct((NP, D), jnp.float32),
        mesh=plsc.VectorSubcoreMesh(core_axis_name="c", subcore_axis_name="s"),
        scratch_types=[
            pltpu.VMEM((TPW,), jnp.int32),
            pltpu.VMEM((TPW,), jnp.int32),
            pltpu.VMEM((TPW, D), jnp.float32),
            pltpu.SemaphoreType.DMA,
            pltpu.SemaphoreType.DMA,
        ],
    )


def _gemm_body(eot_ref, ntot_ref, x_ref, wg_ref, wu_ref, w2_ref, o_ref):
    i = pl.program_id(0)
    c = pl.program_id(1)

    @pl.when(i < ntot_ref[0])
    def _():
        x = x_ref[...]
        gate = lax.dot_general(x, wg_ref[0], (((1,), (1,)), ((), ())),
                               preferred_element_type=jnp.float32)
        up = lax.dot_general(x, wu_ref[0], (((1,), (1,)), ((), ())),
                             preferred_element_type=jnp.float32)
        h = gate * jax.nn.sigmoid(gate) * up
        part = lax.dot_general(h, w2_ref[0], (((1,), (1,)), ((), ())),
                               preferred_element_type=jnp.float32)

        @pl.when(c == 0)
        def _():
            o_ref[...] = part

        @pl.when(c > 0)
        def _():
            o_ref[...] += part


def _gemm(x_sorted, w13_weight, w2_weight, eot, ntot):
    def _ce(i, c, ntot):
        return jnp.where(i < ntot[0], c, C - 1)

    grid_spec = pltpu.PrefetchScalarGridSpec(
        num_scalar_prefetch=2,
        grid=(NT, C),
        in_specs=[
            pl.BlockSpec((TILE, D),
                         lambda i, c, eot, ntot: (jnp.minimum(i, ntot[0] - 1), 0)),
            pl.BlockSpec((1, CF, D),
                         lambda i, c, eot, ntot: (eot[i], _ce(i, c, ntot), 0)),
            pl.BlockSpec((1, CF, D),
                         lambda i, c, eot, ntot: (eot[i], C + _ce(i, c, ntot), 0)),
            pl.BlockSpec((1, D, CF),
                         lambda i, c, eot, ntot: (eot[i], 0, _ce(i, c, ntot))),
        ],
        out_specs=pl.BlockSpec((TILE, D), lambda i, c, eot, ntot: (i, 0)),
    )
    return pl.pallas_call(
        _gemm_body,
        grid_spec=grid_spec,
        out_shape=jax.ShapeDtypeStruct((NP, D), jnp.float32),
        compiler_params=pltpu.CompilerParams(
            dimension_semantics=("arbitrary", "arbitrary"),
            vmem_limit_bytes=100 * 1024 * 1024,
        ),
    )(eot, ntot, x_sorted, w13_weight, w13_weight, w2_weight)


def _combine_body(os_hbm, s0_hbm, s1_hbm, w0_hbm, w1_hbm, out_hbm,
                  idx0, idx1, wv0, wv1, buf_a, buf_b, sem_a, sem_b):
    wid = lax.axis_index("c") * (NW // 2) + lax.axis_index("s")
    base = wid * TPW
    pltpu.sync_copy(s0_hbm.at[pl.ds(base, TPW)], idx0)
    pltpu.sync_copy(s1_hbm.at[pl.ds(base, TPW)], idx1)
    pltpu.sync_copy(w0_hbm.at[pl.ds(base, TPW)], wv0)
    pltpu.sync_copy(w1_hbm.at[pl.ds(base, TPW)], wv1)
    ca = pltpu.async_copy(os_hbm.at[idx0], buf_a, sem_a)
    cb = pltpu.async_copy(os_hbm.at[idx1], buf_b, sem_b)
    ca.wait()
    cb.wait()

    def row(r, carry):
        wa = wv0[r, :]
        wb = wv1[r, :]
        for c in range(D // LANES):
            sl = pl.ds(c * LANES, LANES)
            buf_a[r, sl] = wa * buf_a[r, sl] + wb * buf_b[r, sl]
        return carry

    lax.fori_loop(0, TPW, row, 0)
    pltpu.sync_copy(buf_a, out_hbm.at[pl.ds(base, TPW)])


@functools.cache
def _make_combine():
    return pl.kernel(
        _combine_body,
        out_type=jax.ShapeDtypeStruct((T, D), jnp.float32),
        mesh=plsc.VectorSubcoreMesh(core_axis_name="c", subcore_axis_name="s"),
        scratch_types=[
            pltpu.VMEM((TPW,), jnp.int32),
            pltpu.VMEM((TPW,), jnp.int32),
            pltpu.VMEM((TPW, LANES), jnp.float32),
            pltpu.VMEM((TPW, LANES), jnp.float32),
            pltpu.VMEM((TPW, D), jnp.float32),
            pltpu.VMEM((TPW, D), jnp.float32),
            pltpu.SemaphoreType.DMA,
            pltpu.SemaphoreType.DMA,
        ],
    )


def kernel(hidden_states, router_logits, w13_weight, w2_weight):
    slot0, slot1, w0, w1, eot, ntot = _routing(router_logits)
    x_sorted = _make_dispatch()(hidden_states, slot0, slot1)
    o_sorted = _gemm(x_sorted, w13_weight, w2_weight, eot, ntot)
    return _make_combine()(o_sorted, slot0, slot1, w0, w1)


# speedup vs baseline: 1.5026x; 1.0745x over previous
import functools

import jax
import jax.numpy as jnp
from jax import lax
from jax.experimental import pallas as pl
from jax.experimental.pallas import tpu as pltpu
from jax.experimental.pallas import tpu_sc as plsc

E = 16
TOPK = 2
T = 2048
D = 768
F = 1536
P = T * TOPK
TILE = 384
NT = P // TILE + (E - 1)
NTPAD = 32
CF = 768
C = F // CF
NP = NT * TILE
NW = 32
TPW = T // NW
LANES = 16


def _routing_body(logits_ref, slot0_ref, slot1_ref, w0_ref, w1_ref,
                  eot_ref, ntot_ref):
    logits = logits_ref[...]
    m = jnp.max(logits, axis=1, keepdims=True)
    ex = jnp.exp(logits - m)
    probs = ex / jnp.sum(ex, axis=1, keepdims=True)

    ie = lax.broadcasted_iota(jnp.int32, (T, E), 1)
    m1 = jnp.max(probs, axis=1, keepdims=True)
    a1 = jnp.min(jnp.where(probs >= m1, ie, E), axis=1, keepdims=True)
    oh1 = (ie == a1)
    pmask = jnp.where(oh1, -1.0, probs)
    m2 = jnp.max(pmask, axis=1, keepdims=True)
    a2 = jnp.min(jnp.where(pmask >= m2, ie, E), axis=1, keepdims=True)
    oh2 = (ie == a2)
    rs = m1 + m2
    w0_ref[...] = jnp.broadcast_to(m1 / rs, (T, LANES))
    w1_ref[...] = jnp.broadcast_to(m2 / rs, (T, LANES))

    oh1f = oh1.astype(jnp.float32)
    oh2f = oh2.astype(jnp.float32)
    bb = oh1f + oh2f
    tri = (lax.broadcasted_iota(jnp.int32, (128, 128), 0)
           > lax.broadcasted_iota(jnp.int32, (128, 128), 1)).astype(jnp.float32)
    run = jnp.zeros((1, E), jnp.float32)
    blocks = []
    for b in range(T // 128):
        blk = bb[b * 128:(b + 1) * 128, :]
        pref = lax.dot_general(tri, blk, (((1,), (0,)), ((), ())),
                               preferred_element_type=jnp.float32)
        blocks.append(pref + run)
        run = run + jnp.sum(blk, axis=0, keepdims=True)
    cex = jnp.concatenate(blocks, axis=0)
    cnt = run.astype(jnp.int32)
    ntile = (cnt + (TILE - 1)) // TILE
    upper = (lax.broadcasted_iota(jnp.int32, (E, E), 0)
             < lax.broadcasted_iota(jnp.int32, (E, E), 1)).astype(jnp.float32)
    ts = lax.dot_general(ntile.astype(jnp.float32), upper,
                         (((1,), (0,)), ((), ())),
                         preferred_element_type=jnp.float32)
    seg = ts * TILE
    total = jnp.sum(ntile)
    slot0_ref[...] = jnp.sum(oh1f * (cex + seg), axis=1).astype(jnp.int32)
    slot1_ref[...] = jnp.sum(oh2f * (cex + seg), axis=1).astype(jnp.int32)
    jm = lax.broadcasted_iota(jnp.int32, (NTPAD, E), 0)
    jj = jnp.minimum(jm, total - 1)
    tsi = ts.astype(jnp.int32)
    eot_ref[...] = jnp.sum((jj >= tsi).astype(jnp.int32), axis=1) - 1
    ntot_ref[0] = total


def _routing(router_logits):
    return pl.pallas_call(
        _routing_body,
        out_shape=(
            jax.ShapeDtypeStruct((T,), jnp.int32),
            jax.ShapeDtypeStruct((T,), jnp.int32),
            jax.ShapeDtypeStruct((T, LANES), jnp.float32),
            jax.ShapeDtypeStruct((T, LANES), jnp.float32),
            jax.ShapeDtypeStruct((NTPAD,), jnp.int32),
            jax.ShapeDtypeStruct((1,), jnp.int32),
        ),
        out_specs=(
            pl.BlockSpec((T,), lambda: (0,)),
            pl.BlockSpec((T,), lambda: (0,)),
            pl.BlockSpec((T, LANES), lambda: (0, 0)),
            pl.BlockSpec((T, LANES), lambda: (0, 0)),
            pl.BlockSpec((NTPAD,), lambda: (0,)),
            pl.BlockSpec(memory_space=pltpu.SMEM),
        ),
    )(router_logits)


def _dispatch_body(h_hbm, s0_hbm, s1_hbm, xs_hbm, idx0, idx1, xbuf,
                   sem0, sem1):
    wid = lax.axis_index("c") * (NW // 2) + lax.axis_index("s")
    base = wid * TPW
    pltpu.sync_copy(s0_hbm.at[pl.ds(base, TPW)], idx0)
    pltpu.sync_copy(s1_hbm.at[pl.ds(base, TPW)], idx1)
    pltpu.sync_copy(h_hbm.at[pl.ds(base, TPW)], xbuf)
    c0 = pltpu.async_copy(xbuf, xs_hbm.at[idx0], sem0)
    c1 = pltpu.async_copy(xbuf, xs_hbm.at[idx1], sem1)
    c0.wait()
    c1.wait()


@functools.cache
def _make_dispatch():
    return pl.kernel(
        _dispatch_body,
        out_type=jax.ShapeDtypeStruct((NP, D), jnp.float32),
        mesh=plsc.VectorSubcoreMesh(core_axis_name="c", subcore_axis_name="s"),
        scratch_types=[
            pltpu.VMEM((TPW,), jnp.int32),
            pltpu.VMEM((TPW,), jnp.int32),
            pltpu.VMEM((TPW, D), jnp.float32),
            pltpu.SemaphoreType.DMA,
            pltpu.SemaphoreType.DMA,
        ],
    )


def _gemm_body(eot_ref, ntot_ref, x_ref, wg_ref, wu_ref, w2_ref, o_ref):
    i = pl.program_id(0)
    c = pl.program_id(1)

    @pl.when(i < ntot_ref[0])
    def _():
        x = x_ref[...]
        gate = lax.dot_general(x, wg_ref[0], (((1,), (1,)), ((), ())),
                               preferred_element_type=jnp.float32)
        up = lax.dot_general(x, wu_ref[0], (((1,), (1,)), ((), ())),
                             preferred_element_type=jnp.float32)
        h = gate * jax.nn.sigmoid(gate) * up
        part = lax.dot_general(h, w2_ref[0], (((1,), (1,)), ((), ())),
                               preferred_element_type=jnp.float32)

        @pl.when(c == 0)
        def _():
            o_ref[...] = part

        @pl.when(c > 0)
        def _():
            o_ref[...] += part


def _gemm(x_sorted, w13_weight, w2_weight, eot, ntot):
    def _ce(i, c, ntot):
        return jnp.where(i < ntot[0], c, C - 1)

    grid_spec = pltpu.PrefetchScalarGridSpec(
        num_scalar_prefetch=2,
        grid=(NT, C),
        in_specs=[
            pl.BlockSpec((TILE, D),
                         lambda i, c, eot, ntot: (jnp.minimum(i, ntot[0] - 1), 0)),
            pl.BlockSpec((1, CF, D),
                         lambda i, c, eot, ntot: (eot[i], _ce(i, c, ntot), 0)),
            pl.BlockSpec((1, CF, D),
                         lambda i, c, eot, ntot: (eot[i], C + _ce(i, c, ntot), 0)),
            pl.BlockSpec((1, D, CF),
                         lambda i, c, eot, ntot: (eot[i], 0, _ce(i, c, ntot))),
        ],
        out_specs=pl.BlockSpec((TILE, D), lambda i, c, eot, ntot: (i, 0)),
    )
    return pl.pallas_call(
        _gemm_body,
        grid_spec=grid_spec,
        out_shape=jax.ShapeDtypeStruct((NP, D), jnp.float32),
        compiler_params=pltpu.CompilerParams(
            dimension_semantics=("arbitrary", "arbitrary"),
            vmem_limit_bytes=100 * 1024 * 1024,
        ),
    )(eot, ntot, x_sorted, w13_weight, w13_weight, w2_weight)


def _combine_body(os_hbm, s0_hbm, s1_hbm, w0_hbm, w1_hbm, out_hbm,
                  idx0, idx1, wv0, wv1, buf_a, buf_b, sem_a, sem_b):
    wid = lax.axis_index("c") * (NW // 2) + lax.axis_index("s")
    base = wid * TPW
    pltpu.sync_copy(s0_hbm.at[pl.ds(base, TPW)], idx0)
    pltpu.sync_copy(s1_hbm.at[pl.ds(base, TPW)], idx1)
    pltpu.sync_copy(w0_hbm.at[pl.ds(base, TPW)], wv0)
    pltpu.sync_copy(w1_hbm.at[pl.ds(base, TPW)], wv1)
    ca = pltpu.async_copy(os_hbm.at[idx0], buf_a, sem_a)
    cb = pltpu.async_copy(os_hbm.at[idx1], buf_b, sem_b)
    ca.wait()
    cb.wait()

    def row(r, carry):
        wa = wv0[r, :]
        wb = wv1[r, :]
        for c in range(D // LANES):
            sl = pl.ds(c * LANES, LANES)
            buf_a[r, sl] = wa * buf_a[r, sl] + wb * buf_b[r, sl]
        return carry

    lax.fori_loop(0, TPW, row, 0)
    pltpu.sync_copy(buf_a, out_hbm.at[pl.ds(base, TPW)])


@functools.cache
def _make_combine():
    return pl.kernel(
        _combine_body,
        out_type=jax.ShapeDtypeStruct((T, D), jnp.float32),
        mesh=plsc.VectorSubcoreMesh(core_axis_name="c", subcore_axis_name="s"),
        scratch_types=[
            pltpu.VMEM((TPW,), jnp.int32),
            pltpu.VMEM((TPW,), jnp.int32),
            pltpu.VMEM((TPW, LANES), jnp.float32),
            pltpu.VMEM((TPW, LANES), jnp.float32),
            pltpu.VMEM((TPW, D), jnp.float32),
            pltpu.VMEM((TPW, D), jnp.float32),
            pltpu.SemaphoreType.DMA,
            pltpu.SemaphoreType.DMA,
        ],
    )


def kernel(hidden_states, router_logits, w13_weight, w2_weight):
    slot0, slot1, w0, w1, eot, ntot = _routing(router_logits)
    x_sorted = _make_dispatch()(hidden_states, slot0, slot1)
    o_sorted = _gemm(x_sorted, w13_weight, w2_weight, eot, ntot)
    return _make_combine()(o_sorted, slot0, slot1, w0, w1)


# speedup vs baseline: 1.6596x; 1.1045x over previous
import functools

import jax
import jax.numpy as jnp
from jax import lax
from jax.experimental import pallas as pl
from jax.experimental.pallas import tpu as pltpu
from jax.experimental.pallas import tpu_sc as plsc

E = 16
TOPK = 2
T = 2048
D = 768
F = 1536
P = T * TOPK
TILE = 384
NT = P // TILE + (E - 1)
NTPAD = 32
CF = 1536
C = F // CF
NP = NT * TILE
NW = 32
TPW = T // NW
LANES = 16


def _routing_body(logits_ref, slot0_ref, slot1_ref, w0_ref, w1_ref,
                  eot_ref, ntot_ref):
    logits = logits_ref[...]
    m = jnp.max(logits, axis=1, keepdims=True)
    ex = jnp.exp(logits - m)
    probs = ex / jnp.sum(ex, axis=1, keepdims=True)

    ie = lax.broadcasted_iota(jnp.int32, (T, E), 1)
    m1 = jnp.max(probs, axis=1, keepdims=True)
    a1 = jnp.min(jnp.where(probs >= m1, ie, E), axis=1, keepdims=True)
    oh1 = (ie == a1)
    pmask = jnp.where(oh1, -1.0, probs)
    m2 = jnp.max(pmask, axis=1, keepdims=True)
    a2 = jnp.min(jnp.where(pmask >= m2, ie, E), axis=1, keepdims=True)
    oh2 = (ie == a2)
    rs = m1 + m2
    w0_ref[...] = jnp.broadcast_to(m1 / rs, (T, LANES))
    w1_ref[...] = jnp.broadcast_to(m2 / rs, (T, LANES))

    oh1f = oh1.astype(jnp.float32)
    oh2f = oh2.astype(jnp.float32)
    bb = oh1f + oh2f
    tri = (lax.broadcasted_iota(jnp.int32, (128, 128), 0)
           > lax.broadcasted_iota(jnp.int32, (128, 128), 1)).astype(jnp.float32)
    run = jnp.zeros((1, E), jnp.float32)
    blocks = []
    for b in range(T // 128):
        blk = bb[b * 128:(b + 1) * 128, :]
        pref = lax.dot_general(tri, blk, (((1,), (0,)), ((), ())),
                               preferred_element_type=jnp.float32)
        blocks.append(pref + run)
        run = run + jnp.sum(blk, axis=0, keepdims=True)
    cex = jnp.concatenate(blocks, axis=0)
    cnt = run.astype(jnp.int32)
    ntile = (cnt + (TILE - 1)) // TILE
    upper = (lax.broadcasted_iota(jnp.int32, (E, E), 0)
             < lax.broadcasted_iota(jnp.int32, (E, E), 1)).astype(jnp.float32)
    ts = lax.dot_general(ntile.astype(jnp.float32), upper,
                         (((1,), (0,)), ((), ())),
                         preferred_element_type=jnp.float32)
    seg = ts * TILE
    total = jnp.sum(ntile)
    slot0_ref[...] = jnp.sum(oh1f * (cex + seg), axis=1).astype(jnp.int32)
    slot1_ref[...] = jnp.sum(oh2f * (cex + seg), axis=1).astype(jnp.int32)
    jm = lax.broadcasted_iota(jnp.int32, (NTPAD, E), 0)
    jj = jnp.minimum(jm, total - 1)
    tsi = ts.astype(jnp.int32)
    eot_ref[...] = jnp.sum((jj >= tsi).astype(jnp.int32), axis=1) - 1
    ntot_ref[0] = total


def _routing(router_logits):
    return pl.pallas_call(
        _routing_body,
        out_shape=(
            jax.ShapeDtypeStruct((T,), jnp.int32),
            jax.ShapeDtypeStruct((T,), jnp.int32),
            jax.ShapeDtypeStruct((T, LANES), jnp.float32),
            jax.ShapeDtypeStruct((T, LANES), jnp.float32),
            jax.ShapeDtypeStruct((NTPAD,), jnp.int32),
            jax.ShapeDtypeStruct((1,), jnp.int32),
        ),
        out_specs=(
            pl.BlockSpec((T,), lambda: (0,)),
            pl.BlockSpec((T,), lambda: (0,)),
            pl.BlockSpec((T, LANES), lambda: (0, 0)),
            pl.BlockSpec((T, LANES), lambda: (0, 0)),
            pl.BlockSpec((NTPAD,), lambda: (0,)),
            pl.BlockSpec(memory_space=pltpu.SMEM),
        ),
    )(router_logits)


def _dispatch_body(h_hbm, s0_hbm, s1_hbm, xs_hbm, idx0, idx1, xbuf,
                   sem0, sem1):
    wid = lax.axis_index("c") * (NW // 2) + lax.axis_index("s")
    base = wid * TPW
    pltpu.sync_copy(s0_hbm.at[pl.ds(base, TPW)], idx0)
    pltpu.sync_copy(s1_hbm.at[pl.ds(base, TPW)], idx1)
    pltpu.sync_copy(h_hbm.at[pl.ds(base, TPW)], xbuf)
    c0 = pltpu.async_copy(xbuf, xs_hbm.at[idx0], sem0)
    c1 = pltpu.async_copy(xbuf, xs_hbm.at[idx1], sem1)
    c0.wait()
    c1.wait()


@functools.cache
def _make_dispatch():
    return pl.kernel(
        _dispatch_body,
        out_type=jax.ShapeDtypeStruct((NP, D), jnp.float32),
        mesh=plsc.VectorSubcoreMesh(core_axis_name="c", subcore_axis_name="s"),
        scratch_types=[
            pltpu.VMEM((TPW,), jnp.int32),
            pltpu.VMEM((TPW,), jnp.int32),
            pltpu.VMEM((TPW, D), jnp.float32),
            pltpu.SemaphoreType.DMA,
            pltpu.SemaphoreType.DMA,
        ],
    )


def _gemm_body(eot_ref, ntot_ref, x_ref, wg_ref, wu_ref, w2_ref, o_ref):
    i = pl.program_id(0)
    c = pl.program_id(1)

    @pl.when(i < ntot_ref[0])
    def _():
        x = x_ref[...]
        gate = lax.dot_general(x, wg_ref[0], (((1,), (1,)), ((), ())),
                               preferred_element_type=jnp.float32)
        up = lax.dot_general(x, wu_ref[0], (((1,), (1,)), ((), ())),
                             preferred_element_type=jnp.float32)
        h = gate * jax.nn.sigmoid(gate) * up
        part = lax.dot_general(h, w2_ref[0], (((1,), (1,)), ((), ())),
                               preferred_element_type=jnp.float32)

        @pl.when(c == 0)
        def _():
            o_ref[...] = part

        @pl.when(c > 0)
        def _():
            o_ref[...] += part


def _gemm(x_sorted, w13_weight, w2_weight, eot, ntot):
    def _ce(i, c, ntot):
        return jnp.where(i < ntot[0], c, C - 1)

    grid_spec = pltpu.PrefetchScalarGridSpec(
        num_scalar_prefetch=2,
        grid=(NT, C),
        in_specs=[
            pl.BlockSpec((TILE, D),
                         lambda i, c, eot, ntot: (jnp.minimum(i, ntot[0] - 1), 0)),
            pl.BlockSpec((1, CF, D),
                         lambda i, c, eot, ntot: (eot[i], _ce(i, c, ntot), 0)),
            pl.BlockSpec((1, CF, D),
                         lambda i, c, eot, ntot: (eot[i], C + _ce(i, c, ntot), 0)),
            pl.BlockSpec((1, D, CF),
                         lambda i, c, eot, ntot: (eot[i], 0, _ce(i, c, ntot))),
        ],
        out_specs=pl.BlockSpec((TILE, D), lambda i, c, eot, ntot: (i, 0)),
    )
    return pl.pallas_call(
        _gemm_body,
        grid_spec=grid_spec,
        out_shape=jax.ShapeDtypeStruct((NP, D), jnp.float32),
        compiler_params=pltpu.CompilerParams(
            dimension_semantics=("arbitrary", "arbitrary"),
            vmem_limit_bytes=100 * 1024 * 1024,
        ),
    )(eot, ntot, x_sorted, w13_weight, w13_weight, w2_weight)


def _combine_body(os_hbm, s0_hbm, s1_hbm, w0_hbm, w1_hbm, out_hbm,
                  idx0, idx1, wv0, wv1, buf_a, buf_b, sem_a, sem_b):
    wid = lax.axis_index("c") * (NW // 2) + lax.axis_index("s")
    base = wid * TPW
    pltpu.sync_copy(s0_hbm.at[pl.ds(base, TPW)], idx0)
    pltpu.sync_copy(s1_hbm.at[pl.ds(base, TPW)], idx1)
    pltpu.sync_copy(w0_hbm.at[pl.ds(base, TPW)], wv0)
    pltpu.sync_copy(w1_hbm.at[pl.ds(base, TPW)], wv1)
    ca = pltpu.async_copy(os_hbm.at[idx0], buf_a, sem_a)
    cb = pltpu.async_copy(os_hbm.at[idx1], buf_b, sem_b)
    ca.wait()
    cb.wait()

    def row(r, carry):
        wa = wv0[r, :]
        wb = wv1[r, :]
        for c in range(D // LANES):
            sl = pl.ds(c * LANES, LANES)
            buf_a[r, sl] = wa * buf_a[r, sl] + wb * buf_b[r, sl]
        return carry

    lax.fori_loop(0, TPW, row, 0)
    pltpu.sync_copy(buf_a, out_hbm.at[pl.ds(base, TPW)])


@functools.cache
def _make_combine():
    return pl.kernel(
        _combine_body,
        out_type=jax.ShapeDtypeStruct((T, D), jnp.float32),
        mesh=plsc.VectorSubcoreMesh(core_axis_name="c", subcore_axis_name="s"),
        scratch_types=[
            pltpu.VMEM((TPW,), jnp.int32),
            pltpu.VMEM((TPW,), jnp.int32),
            pltpu.VMEM((TPW, LANES), jnp.float32),
            pltpu.VMEM((TPW, LANES), jnp.float32),
            pltpu.VMEM((TPW, D), jnp.float32),
            pltpu.VMEM((TPW, D), jnp.float32),
            pltpu.SemaphoreType.DMA,
            pltpu.SemaphoreType.DMA,
        ],
    )


def kernel(hidden_states, router_logits, w13_weight, w2_weight):
    slot0, slot1, w0, w1, eot, ntot = _routing(router_logits)
    x_sorted = _make_dispatch()(hidden_states, slot0, slot1)
    o_sorted = _gemm(x_sorted, w13_weight, w2_weight, eot, ntot)
    return _make_combine()(o_sorted, slot0, slot1, w0, w1)
